# baseline (speedup 1.0000x reference)
import jax
import jax.numpy as jnp
from jax import lax
from jax.experimental import pallas as pl
from jax.experimental.pallas import tpu as pltpu
from jax.experimental.pallas import tpu_sc as plsc

B = 16384
EMB = 64
H = 128
NC = 2
NS = 16
NW = NC * NS
BPW = B // NW

BLK = 2048


def _sc_gather_body(user_table, item_table, uid, iid, u_out, v_out,
                    idx_u, idx_v, rows_u, rows_v, sem_u, sem_v):
    wid = lax.axis_index("s") * NC + lax.axis_index("c")
    base = wid * BPW
    pltpu.sync_copy(uid.at[pl.ds(base, BPW)], idx_u)
    pltpu.sync_copy(iid.at[pl.ds(base, BPW)], idx_v)
    cu = pltpu.async_copy(user_table.at[idx_u], rows_u, sem_u)
    cv = pltpu.async_copy(item_table.at[idx_v], rows_v, sem_v)
    cu.wait()
    pltpu.sync_copy(rows_u, u_out.at[pl.ds(base, BPW)])
    cv.wait()
    pltpu.sync_copy(rows_v, v_out.at[pl.ds(base, BPW)])


_sc_gather = pl.kernel(
    _sc_gather_body,
    mesh=plsc.VectorSubcoreMesh(core_axis_name="c", subcore_axis_name="s"),
    out_type=[
        jax.ShapeDtypeStruct((B, EMB), jnp.bfloat16),
        jax.ShapeDtypeStruct((B, EMB), jnp.bfloat16),
    ],
    scratch_types=[
        pltpu.VMEM((BPW,), jnp.int32),
        pltpu.VMEM((BPW,), jnp.int32),
        pltpu.VMEM((BPW, EMB), jnp.bfloat16),
        pltpu.VMEM((BPW, EMB), jnp.bfloat16),
        pltpu.SemaphoreType.DMA,
        pltpu.SemaphoreType.DMA,
    ],
    compiler_params=pltpu.CompilerParams(use_tc_tiling_on_sc=False),
)


def _tc_towers_body(u_ref, v_ref, w0u, b0u, w1u, b1u, w0i, b0i, w1i, b1i,
                    o_ref):
    def tower(x, W0, b0, W1, b1):
        h = lax.dot_general(x, W0[...], (((1,), (1,)), ((), ())),
                            preferred_element_type=jnp.float32)
        h = jnp.maximum(h + b0[...], 0.0)
        h = lax.dot_general(h, W1[...], (((1,), (1,)), ((), ())),
                            preferred_element_type=jnp.float32)
        return jnp.maximum(h + b1[...], 0.0)

    uo = tower(u_ref[...].astype(jnp.float32), w0u, b0u, w1u, b1u)
    vo = tower(v_ref[...].astype(jnp.float32), w0i, b0i, w1i, b1i)
    o_ref[...] = jnp.sum(uo * vo, axis=-1)


def _tc_towers(u_rows, v_rows, W0_u, b0_u, W1_u, b1_u, W0_i, b0_i, W1_i, b1_i):
    full = lambda shape: pl.BlockSpec(shape, lambda i: (0,) * len(shape))
    return pl.pallas_call(
        _tc_towers_body,
        grid=(B // BLK,),
        in_specs=[
            pl.BlockSpec((BLK, EMB), lambda i: (i, 0)),
            pl.BlockSpec((BLK, EMB), lambda i: (i, 0)),
            full((H, EMB)), full((1, H)),
            full((EMB, H)), full((1, EMB)),
            full((H, EMB)), full((1, H)),
            full((EMB, H)), full((1, EMB)),
        ],
        out_specs=pl.BlockSpec((BLK,), lambda i: (i,)),
        out_shape=jax.ShapeDtypeStruct((B,), jnp.float32),
    )(u_rows, v_rows, W0_u, b0_u.reshape(1, H), W1_u, b1_u.reshape(1, EMB),
      W0_i, b0_i.reshape(1, H), W1_i, b1_i.reshape(1, EMB))


def kernel(user_ids, item_ids, user_table, item_table,
           W0_u, b0_u, W1_u, b1_u, W0_i, b0_i, W1_i, b1_i):
    uid = user_ids.astype(jnp.int32)
    iid = item_ids.astype(jnp.int32)
    ub = user_table.astype(jnp.bfloat16)
    ib = item_table.astype(jnp.bfloat16)
    u_rows, v_rows = _sc_gather(ub, ib, uid, iid)
    return _tc_towers(u_rows, v_rows, W0_u, b0_u, W1_u, b1_u,
                      W0_i, b0_i, W1_i, b1_i)


# speedup vs baseline: 1.7032x; 1.7032x over previous
import jax
import jax.numpy as jnp
from jax import lax
from jax.experimental import pallas as pl
from jax.experimental.pallas import tpu as pltpu
from jax.experimental.pallas import tpu_sc as plsc

B = 16384
EMB = 64
H = 128
N_ROWS = 1000000
NC = 2
NS = 16
NW = NC * NS
BPW = B // NW

RBLK = 512
NBLK = 489
R = NBLK * RBLK

BLK = 2048

_MAX_CBLK = (N_ROWS + RBLK - 1) // RBLK - 1
_HI = -65536


def _repack_body(t0, t1, t2, t3, o_ref):
    def rows(tk):
        b = lax.bitcast_convert_type(jnp.swapaxes(tk[...], 0, 1), jnp.int32)
        return (b + 32768) & _HI

    r0, r1, r2, r3 = rows(t0), rows(t1), rows(t2), rows(t3)
    o_ref[:, :EMB] = r0 | lax.shift_right_logical(r2, 16)
    o_ref[:, EMB:] = r1 | lax.shift_right_logical(r3, 16)


def _repack(table_t):
    view = lambda k: pl.BlockSpec(
        (EMB, RBLK), lambda i, k=k: (0, jnp.minimum(k * NBLK + i, _MAX_CBLK)))
    return pl.pallas_call(
        _repack_body,
        grid=(NBLK,),
        in_specs=[view(0), view(1), view(2), view(3)],
        out_specs=pl.BlockSpec((RBLK, H), lambda i: (i, 0)),
        out_shape=jax.ShapeDtypeStruct((R, H), jnp.int32),
    )(table_t, table_t, table_t, table_t)


def _sc_gather_body(pu, pi, uid, iid, u_out, v_out,
                    idx_raw, idx_p, rows, sem):
    wid = lax.axis_index("s") * NC + lax.axis_index("c")
    base = wid * BPW
    for ids, packed, out in ((uid, pu, u_out), (iid, pi, v_out)):
        pltpu.sync_copy(ids.at[pl.ds(base, BPW)], idx_raw)

        def to_packed(g, _):
            v = idx_raw[pl.ds(g * 16, 16)]
            idx_p[pl.ds(g * 16, 16)] = v - lax.div(v, R) * R
            return ()

        lax.fori_loop(0, BPW // 16, to_packed, ())
        pltpu.async_copy(packed.at[idx_p], rows, sem).wait()
        pltpu.sync_copy(rows, out.at[pl.ds(base, BPW)])


_sc_gather = pl.kernel(
    _sc_gather_body,
    mesh=plsc.VectorSubcoreMesh(core_axis_name="c", subcore_axis_name="s"),
    out_type=[
        jax.ShapeDtypeStruct((B, H), jnp.int32),
        jax.ShapeDtypeStruct((B, H), jnp.int32),
    ],
    scratch_types=[
        pltpu.VMEM((BPW,), jnp.int32),
        pltpu.VMEM((BPW,), jnp.int32),
        pltpu.VMEM((BPW, H), jnp.int32),
        pltpu.SemaphoreType.DMA,
    ],
    compiler_params=pltpu.CompilerParams(use_tc_tiling_on_sc=True),
)


def _tc_towers_body(u_ref, v_ref, uid, iid, w0u, b0u, w1u, b1u,
                    w0i, b0i, w1i, b1i, o_ref):
    def decode(q_ref, ids_ref):
        q = q_ref[...]
        k = (ids_ref[...] // R)[:, None]
        w = jnp.where(k % 2 == 0, q[:, :EMB], q[:, EMB:])
        bits = jnp.where(k < 2, w & _HI, lax.shift_left(w, 16))
        return lax.bitcast_convert_type(bits, jnp.float32)

    def tower(x, W0, b0, W1, b1):
        h = lax.dot_general(x, W0[...], (((1,), (1,)), ((), ())),
                            preferred_element_type=jnp.float32)
        h = jnp.maximum(h + b0[...], 0.0)
        h = lax.dot_general(h, W1[...], (((1,), (1,)), ((), ())),
                            preferred_element_type=jnp.float32)
        return jnp.maximum(h + b1[...], 0.0)

    uo = tower(decode(u_ref, uid), w0u, b0u, w1u, b1u)
    vo = tower(decode(v_ref, iid), w0i, b0i, w1i, b1i)
    o_ref[...] = jnp.sum(uo * vo, axis=-1)


def _tc_towers(u_r, v_r, uid, iid,
               W0_u, b0_u, W1_u, b1_u, W0_i, b0_i, W1_i, b1_i):
    full = lambda shape: pl.BlockSpec(shape, lambda i: (0,) * len(shape))
    return pl.pallas_call(
        _tc_towers_body,
        grid=(B // BLK,),
        in_specs=[
            pl.BlockSpec((BLK, H), lambda i: (i, 0)),
            pl.BlockSpec((BLK, H), lambda i: (i, 0)),
            pl.BlockSpec((BLK,), lambda i: (i,)),
            pl.BlockSpec((BLK,), lambda i: (i,)),
            full((H, EMB)), full((1, H)),
            full((EMB, H)), full((1, EMB)),
            full((H, EMB)), full((1, H)),
            full((EMB, H)), full((1, EMB)),
        ],
        out_specs=pl.BlockSpec((BLK,), lambda i: (i,)),
        out_shape=jax.ShapeDtypeStruct((B,), jnp.float32),
    )(u_r, v_r, uid, iid,
      W0_u, b0_u.reshape(1, H), W1_u, b1_u.reshape(1, EMB),
      W0_i, b0_i.reshape(1, H), W1_i, b1_i.reshape(1, EMB))


def kernel(user_ids, item_ids, user_table, item_table,
           W0_u, b0_u, W1_u, b1_u, W0_i, b0_i, W1_i, b1_i):
    uid = user_ids.astype(jnp.int32)
    iid = item_ids.astype(jnp.int32)
    pu = _repack(user_table.T)
    pi = _repack(item_table.T)
    u_r, v_r = _sc_gather(pu, pi, uid, iid)
    return _tc_towers(u_r, v_r, uid, iid, W0_u, b0_u, W1_u, b1_u,
                      W0_i, b0_i, W1_i, b1_i)


# speedup vs baseline: 3.0105x; 1.7676x over previous
import jax
import jax.numpy as jnp
from jax import lax
from jax.experimental import pallas as pl
from jax.experimental.pallas import tpu as pltpu
from jax.experimental.pallas import tpu_sc as plsc

B = 16384
EMB = 64
H = 128
N_ROWS = 1000000
NC = 2
NS = 16
NW = NC * NS
BPW = B // NW

RBLK = 2048
NBLK = 123
R = NBLK * RBLK

BLK = 2048

_MAX_CBLK = (N_ROWS + RBLK - 1) // RBLK - 1
_HI = -65536


def _repack_body(t0, t1, t2, t3, eye, o_ref):
    def rows(tk):
        t = lax.dot_general(tk[...], eye[...], (((0,), (0,)), ((), ())),
                            preferred_element_type=jnp.float32)
        b = lax.bitcast_convert_type(t, jnp.int32)
        return (b + 32768) & _HI

    r0, r1, r2, r3 = rows(t0), rows(t1), rows(t2), rows(t3)
    o_ref[:, :EMB] = r0 | lax.shift_right_logical(r2, 16)
    o_ref[:, EMB:] = r1 | lax.shift_right_logical(r3, 16)


def _repack(table_t, eye):
    view = lambda k: pl.BlockSpec(
        (EMB, RBLK), lambda i, k=k: (0, jnp.minimum(k * NBLK + i, _MAX_CBLK)))
    return pl.pallas_call(
        _repack_body,
        grid=(NBLK,),
        in_specs=[view(0), view(1), view(2), view(3),
                  pl.BlockSpec((EMB, EMB), lambda i: (0, 0))],
        out_specs=pl.BlockSpec((RBLK, H), lambda i: (i, 0)),
        out_shape=jax.ShapeDtypeStruct((R, H), jnp.int32),
    )(table_t, table_t, table_t, table_t, eye)


def _sc_gather_body(pu, pi, uid, iid, u_out, v_out,
                    idx_raw, idx_p, rows, sem):
    wid = lax.axis_index("s") * NC + lax.axis_index("c")
    base = wid * BPW
    for ids, packed, out in ((uid, pu, u_out), (iid, pi, v_out)):
        pltpu.sync_copy(ids.at[pl.ds(base, BPW)], idx_raw)

        def to_packed(g, _):
            v = idx_raw[pl.ds(g * 16, 16)]
            idx_p[pl.ds(g * 16, 16)] = v - lax.div(v, R) * R
            return ()

        lax.fori_loop(0, BPW // 16, to_packed, ())
        pltpu.async_copy(packed.at[idx_p], rows, sem).wait()
        pltpu.sync_copy(rows, out.at[pl.ds(base, BPW)])


_sc_gather = pl.kernel(
    _sc_gather_body,
    mesh=plsc.VectorSubcoreMesh(core_axis_name="c", subcore_axis_name="s"),
    out_type=[
        jax.ShapeDtypeStruct((B, H), jnp.int32),
        jax.ShapeDtypeStruct((B, H), jnp.int32),
    ],
    scratch_types=[
        pltpu.VMEM((BPW,), jnp.int32),
        pltpu.VMEM((BPW,), jnp.int32),
        pltpu.VMEM((BPW, H), jnp.int32),
        pltpu.SemaphoreType.DMA,
    ],
    compiler_params=pltpu.CompilerParams(use_tc_tiling_on_sc=True),
)


def _tc_towers_body(u_ref, v_ref, uid, iid, w0u, b0u, w1u, b1u,
                    w0i, b0i, w1i, b1i, o_ref):
    def decode(q_ref, ids_ref):
        q = q_ref[...]
        k = (ids_ref[...] // R)[:, None]
        w = jnp.where(k % 2 == 0, q[:, :EMB], q[:, EMB:])
        bits = jnp.where(k < 2, w & _HI, lax.shift_left(w, 16))
        return lax.bitcast_convert_type(bits, jnp.float32)

    def tower(x, W0, b0, W1, b1):
        h = lax.dot_general(x, W0[...], (((1,), (1,)), ((), ())),
                            preferred_element_type=jnp.float32)
        h = jnp.maximum(h + b0[...], 0.0)
        h = lax.dot_general(h, W1[...], (((1,), (1,)), ((), ())),
                            preferred_element_type=jnp.float32)
        return jnp.maximum(h + b1[...], 0.0)

    uo = tower(decode(u_ref, uid), w0u, b0u, w1u, b1u)
    vo = tower(decode(v_ref, iid), w0i, b0i, w1i, b1i)
    o_ref[...] = jnp.sum(uo * vo, axis=-1)


def _tc_towers(u_r, v_r, uid, iid,
               W0_u, b0_u, W1_u, b1_u, W0_i, b0_i, W1_i, b1_i):
    full = lambda shape: pl.BlockSpec(shape, lambda i: (0,) * len(shape))
    return pl.pallas_call(
        _tc_towers_body,
        grid=(B // BLK,),
        in_specs=[
            pl.BlockSpec((BLK, H), lambda i: (i, 0)),
            pl.BlockSpec((BLK, H), lambda i: (i, 0)),
            pl.BlockSpec((BLK,), lambda i: (i,)),
            pl.BlockSpec((BLK,), lambda i: (i,)),
            full((H, EMB)), full((1, H)),
            full((EMB, H)), full((1, EMB)),
            full((H, EMB)), full((1, H)),
            full((EMB, H)), full((1, EMB)),
        ],
        out_specs=pl.BlockSpec((BLK,), lambda i: (i,)),
        out_shape=jax.ShapeDtypeStruct((B,), jnp.float32),
    )(u_r, v_r, uid, iid,
      W0_u, b0_u.reshape(1, H), W1_u, b1_u.reshape(1, EMB),
      W0_i, b0_i.reshape(1, H), W1_i, b1_i.reshape(1, EMB))


def kernel(user_ids, item_ids, user_table, item_table,
           W0_u, b0_u, W1_u, b1_u, W0_i, b0_i, W1_i, b1_i):
    uid = user_ids.astype(jnp.int32)
    iid = item_ids.astype(jnp.int32)
    eye = jnp.eye(EMB, dtype=jnp.float32)
    pu = _repack(user_table.T, eye)
    pi = _repack(item_table.T, eye)
    u_r, v_r = _sc_gather(pu, pi, uid, iid)
    return _tc_towers(u_r, v_r, uid, iid, W0_u, b0_u, W1_u, b1_u,
                      W0_i, b0_i, W1_i, b1_i)


# speedup vs baseline: 3.4379x; 1.1420x over previous
import jax
import jax.numpy as jnp
from jax import lax
from jax.experimental import pallas as pl
from jax.experimental.pallas import tpu as pltpu
from jax.experimental.pallas import tpu_sc as plsc

B = 16384
EMB = 64
H = 128
N_ROWS = 1000000
NC = 2
NS = 16
NW = NC * NS
BPW = B // NW

RBLK = 2048
NBLK = 123
R = NBLK * RBLK

BLK = 2048

_MAX_CBLK = (N_ROWS + RBLK - 1) // RBLK - 1
_HI = -65536


def _repack_body(t0, t1, t2, t3, o_ref):
    def bits(tk):
        b = lax.bitcast_convert_type(tk[...], jnp.int32)
        return (b + 32768) & _HI

    c02 = bits(t0) | lax.shift_right_logical(bits(t2), 16)
    c13 = bits(t1) | lax.shift_right_logical(bits(t3), 16)
    o_ref[:, :EMB] = jnp.swapaxes(c02, 0, 1)
    o_ref[:, EMB:] = jnp.swapaxes(c13, 0, 1)


def _repack(table_t):
    view = lambda k: pl.BlockSpec(
        (EMB, RBLK), lambda i, k=k: (0, jnp.minimum(k * NBLK + i, _MAX_CBLK)))
    return pl.pallas_call(
        _repack_body,
        grid=(NBLK,),
        in_specs=[view(0), view(1), view(2), view(3)],
        out_specs=pl.BlockSpec((RBLK, H), lambda i: (i, 0)),
        out_shape=jax.ShapeDtypeStruct((R, H), jnp.int32),
    )(table_t, table_t, table_t, table_t)


def _sc_gather_body(pu, pi, uid, iid, u_out, v_out,
                    idx_raw, idx_p, rows, sem):
    wid = lax.axis_index("s") * NC + lax.axis_index("c")
    base = wid * BPW
    for ids, packed, out in ((uid, pu, u_out), (iid, pi, v_out)):
        pltpu.sync_copy(ids.at[pl.ds(base, BPW)], idx_raw)

        def to_packed(g, _):
            v = idx_raw[pl.ds(g * 16, 16)]
            idx_p[pl.ds(g * 16, 16)] = v - lax.div(v, R) * R
            return ()

        lax.fori_loop(0, BPW // 16, to_packed, ())
        pltpu.async_copy(packed.at[idx_p], rows, sem).wait()
        pltpu.sync_copy(rows, out.at[pl.ds(base, BPW)])


_sc_gather = pl.kernel(
    _sc_gather_body,
    mesh=plsc.VectorSubcoreMesh(core_axis_name="c", subcore_axis_name="s"),
    out_type=[
        jax.ShapeDtypeStruct((B, H), jnp.int32),
        jax.ShapeDtypeStruct((B, H), jnp.int32),
    ],
    scratch_types=[
        pltpu.VMEM((BPW,), jnp.int32),
        pltpu.VMEM((BPW,), jnp.int32),
        pltpu.VMEM((BPW, H), jnp.int32),
        pltpu.SemaphoreType.DMA,
    ],
    compiler_params=pltpu.CompilerParams(use_tc_tiling_on_sc=True),
)


def _tc_towers_body(u_ref, v_ref, uid, iid, w0u, b0u, w1u, b1u,
                    w0i, b0i, w1i, b1i, o_ref):
    def decode(q_ref, ids_ref):
        q = q_ref[...]
        k = (ids_ref[...] // R)[:, None]
        w = jnp.where(k % 2 == 0, q[:, :EMB], q[:, EMB:])
        bits = jnp.where(k < 2, w & _HI, lax.shift_left(w, 16))
        return lax.bitcast_convert_type(bits, jnp.float32)

    def tower(x, W0, b0, W1, b1):
        h = lax.dot_general(x, W0[...], (((1,), (1,)), ((), ())),
                            preferred_element_type=jnp.float32)
        h = jnp.maximum(h + b0[...], 0.0)
        h = lax.dot_general(h, W1[...], (((1,), (1,)), ((), ())),
                            preferred_element_type=jnp.float32)
        return jnp.maximum(h + b1[...], 0.0)

    uo = tower(decode(u_ref, uid), w0u, b0u, w1u, b1u)
    vo = tower(decode(v_ref, iid), w0i, b0i, w1i, b1i)
    o_ref[...] = jnp.sum(uo * vo, axis=-1)


def _tc_towers(u_r, v_r, uid, iid,
               W0_u, b0_u, W1_u, b1_u, W0_i, b0_i, W1_i, b1_i):
    full = lambda shape: pl.BlockSpec(shape, lambda i: (0,) * len(shape))
    return pl.pallas_call(
        _tc_towers_body,
        grid=(B // BLK,),
        in_specs=[
            pl.BlockSpec((BLK, H), lambda i: (i, 0)),
            pl.BlockSpec((BLK, H), lambda i: (i, 0)),
            pl.BlockSpec((BLK,), lambda i: (i,)),
            pl.BlockSpec((BLK,), lambda i: (i,)),
            full((H, EMB)), full((1, H)),
            full((EMB, H)), full((1, EMB)),
            full((H, EMB)), full((1, H)),
            full((EMB, H)), full((1, EMB)),
        ],
        out_specs=pl.BlockSpec((BLK,), lambda i: (i,)),
        out_shape=jax.ShapeDtypeStruct((B,), jnp.float32),
    )(u_r, v_r, uid, iid,
      W0_u, b0_u.reshape(1, H), W1_u, b1_u.reshape(1, EMB),
      W0_i, b0_i.reshape(1, H), W1_i, b1_i.reshape(1, EMB))


def kernel(user_ids, item_ids, user_table, item_table,
           W0_u, b0_u, W1_u, b1_u, W0_i, b0_i, W1_i, b1_i):
    uid = user_ids.astype(jnp.int32)
    iid = item_ids.astype(jnp.int32)
    pu = _repack(user_table.T)
    pi = _repack(item_table.T)
    u_r, v_r = _sc_gather(pu, pi, uid, iid)
    return _tc_towers(u_r, v_r, uid, iid, W0_u, b0_u, W1_u, b1_u,
                      W0_i, b0_i, W1_i, b1_i)


# speedup vs baseline: 4.2089x; 1.2243x over previous
import jax
import jax.numpy as jnp
from jax import lax
from jax.experimental import pallas as pl
from jax.experimental.pallas import tpu as pltpu
from jax.experimental.pallas import tpu_sc as plsc

B = 16384
EMB = 64
H = 128
N_ROWS = 1000000
NC = 2
NS = 16
NW = NC * NS
BPW = B // NW

RBLK = 2048
NBLK = 123
R = NBLK * RBLK

BLK = 2048

_MAX_CBLK = (N_ROWS + RBLK - 1) // RBLK - 1
_HI = -65536


def _repack_body(u0, u1, u2, u3, i0, i1, i2, i3, ou_ref, oi_ref):
    def bits(tk):
        b = lax.bitcast_convert_type(tk[...], jnp.int32)
        return (b + 32768) & _HI

    for t0, t1, t2, t3, o_ref in ((u0, u1, u2, u3, ou_ref),
                                  (i0, i1, i2, i3, oi_ref)):
        c02 = bits(t0) | lax.shift_right_logical(bits(t2), 16)
        c13 = bits(t1) | lax.shift_right_logical(bits(t3), 16)
        o_ref[:, :EMB] = jnp.swapaxes(c02, 0, 1)
        o_ref[:, EMB:] = jnp.swapaxes(c13, 0, 1)


def _repack(ut, it):
    view = lambda k: pl.BlockSpec(
        (EMB, RBLK), lambda i, k=k: (0, jnp.minimum(k * NBLK + i, _MAX_CBLK)))
    out_spec = pl.BlockSpec((RBLK, H), lambda i: (i, 0))
    out_sds = jax.ShapeDtypeStruct((R, H), jnp.int32)
    return pl.pallas_call(
        _repack_body,
        grid=(NBLK,),
        in_specs=[view(0), view(1), view(2), view(3)] * 2,
        out_specs=[out_spec, out_spec],
        out_shape=[out_sds, out_sds],
    )(ut, ut, ut, ut, it, it, it, it)


def _sc_gather_body(pu, pi, uid, iid, u_out, v_out,
                    idx_raw, idx_p, rows, sem):
    wid = lax.axis_index("s") * NC + lax.axis_index("c")
    base = wid * BPW
    for ids, packed, out in ((uid, pu, u_out), (iid, pi, v_out)):
        pltpu.sync_copy(ids.at[pl.ds(base, BPW)], idx_raw)

        def to_packed(g, _):
            v = idx_raw[pl.ds(g * 16, 16)]
            idx_p[pl.ds(g * 16, 16)] = v - lax.div(v, R) * R
            return ()

        lax.fori_loop(0, BPW // 16, to_packed, ())
        pltpu.async_copy(packed.at[idx_p], rows, sem).wait()
        pltpu.sync_copy(rows, out.at[pl.ds(base, BPW)])


_sc_gather = pl.kernel(
    _sc_gather_body,
    mesh=plsc.VectorSubcoreMesh(core_axis_name="c", subcore_axis_name="s"),
    out_type=[
        jax.ShapeDtypeStruct((B, H), jnp.int32),
        jax.ShapeDtypeStruct((B, H), jnp.int32),
    ],
    scratch_types=[
        pltpu.VMEM((BPW,), jnp.int32),
        pltpu.VMEM((BPW,), jnp.int32),
        pltpu.VMEM((BPW, H), jnp.int32),
        pltpu.SemaphoreType.DMA,
    ],
    compiler_params=pltpu.CompilerParams(use_tc_tiling_on_sc=True),
)


def _tc_towers_body(u_ref, v_ref, uid, iid, w0u, b0u, w1u, b1u,
                    w0i, b0i, w1i, b1i, o_ref):
    def decode(q_ref, ids_ref):
        q = q_ref[...]
        k = (ids_ref[...] // R)[:, None]
        w = jnp.where(k % 2 == 0, q[:, :EMB], q[:, EMB:])
        bits = jnp.where(k < 2, w & _HI, lax.shift_left(w, 16))
        return lax.bitcast_convert_type(bits, jnp.float32)

    def tower(x, W0, b0, W1, b1):
        h = lax.dot_general(x, W0[...], (((1,), (1,)), ((), ())),
                            preferred_element_type=jnp.float32)
        h = jnp.maximum(h + b0[...], 0.0)
        h = lax.dot_general(h, W1[...], (((1,), (1,)), ((), ())),
                            preferred_element_type=jnp.float32)
        return jnp.maximum(h + b1[...], 0.0)

    uo = tower(decode(u_ref, uid), w0u, b0u, w1u, b1u)
    vo = tower(decode(v_ref, iid), w0i, b0i, w1i, b1i)
    o_ref[...] = jnp.sum(uo * vo, axis=-1)


def _tc_towers(u_r, v_r, uid, iid,
               W0_u, b0_u, W1_u, b1_u, W0_i, b0_i, W1_i, b1_i):
    full = lambda shape: pl.BlockSpec(shape, lambda i: (0,) * len(shape))
    return pl.pallas_call(
        _tc_towers_body,
        grid=(B // BLK,),
        in_specs=[
            pl.BlockSpec((BLK, H), lambda i: (i, 0)),
            pl.BlockSpec((BLK, H), lambda i: (i, 0)),
            pl.BlockSpec((BLK,), lambda i: (i,)),
            pl.BlockSpec((BLK,), lambda i: (i,)),
            full((H, EMB)), full((1, H)),
            full((EMB, H)), full((1, EMB)),
            full((H, EMB)), full((1, H)),
            full((EMB, H)), full((1, EMB)),
        ],
        out_specs=pl.BlockSpec((BLK,), lambda i: (i,)),
        out_shape=jax.ShapeDtypeStruct((B,), jnp.float32),
    )(u_r, v_r, uid, iid,
      W0_u, b0_u.reshape(1, H), W1_u, b1_u.reshape(1, EMB),
      W0_i, b0_i.reshape(1, H), W1_i, b1_i.reshape(1, EMB))


def kernel(user_ids, item_ids, user_table, item_table,
           W0_u, b0_u, W1_u, b1_u, W0_i, b0_i, W1_i, b1_i):
    uid = user_ids.astype(jnp.int32)
    iid = item_ids.astype(jnp.int32)
    pu, pi = _repack(user_table.T, item_table.T)
    u_r, v_r = _sc_gather(pu, pi, uid, iid)
    return _tc_towers(u_r, v_r, uid, iid, W0_u, b0_u, W1_u, b1_u,
                      W0_i, b0_i, W1_i, b1_i)


# speedup vs baseline: 4.6995x; 1.1165x over previous
import jax
import jax.numpy as jnp
from jax import lax
from jax.experimental import pallas as pl
from jax.experimental.pallas import tpu as pltpu
from jax.experimental.pallas import tpu_sc as plsc

B = 16384
EMB = 64
H = 128
N_ROWS = 1000000
NC = 2
NS = 16
NW = NC * NS
BPW = B // NW

RBLK = 4096
NBLK = 62
R = NBLK * RBLK

BLK = 2048

_MAX_CBLK = (N_ROWS + RBLK - 1) // RBLK - 1
_HI = -65536


def _repack_body(u0, u1, u2, u3, i0, i1, i2, i3, ou_ref, oi_ref):
    def bits(tk):
        b = lax.bitcast_convert_type(tk[...], jnp.int32)
        return (b + 32768) & _HI

    for t0, t1, t2, t3, o_ref in ((u0, u1, u2, u3, ou_ref),
                                  (i0, i1, i2, i3, oi_ref)):
        c02 = bits(t0) | lax.shift_right_logical(bits(t2), 16)
        c13 = bits(t1) | lax.shift_right_logical(bits(t3), 16)
        o_ref[:, :EMB] = jnp.swapaxes(c02, 0, 1)
        o_ref[:, EMB:] = jnp.swapaxes(c13, 0, 1)


def _repack(ut, it):
    view = lambda k: pl.BlockSpec(
        (EMB, RBLK), lambda i, k=k: (0, jnp.minimum(k * NBLK + i, _MAX_CBLK)))
    out_spec = pl.BlockSpec((RBLK, H), lambda i: (i, 0))
    out_sds = jax.ShapeDtypeStruct((R, H), jnp.int32)
    return pl.pallas_call(
        _repack_body,
        grid=(NBLK,),
        in_specs=[view(0), view(1), view(2), view(3)] * 2,
        out_specs=[out_spec, out_spec],
        out_shape=[out_sds, out_sds],
    )(ut, ut, ut, ut, it, it, it, it)


def _sc_gather_body(pu, pi, uid, iid, u_out, v_out,
                    idx_raw, idx_p, rows, sem):
    wid = lax.axis_index("s") * NC + lax.axis_index("c")
    base = wid * BPW
    for ids, packed, out in ((uid, pu, u_out), (iid, pi, v_out)):
        pltpu.sync_copy(ids.at[pl.ds(base, BPW)], idx_raw)

        def to_packed(g, _):
            v = idx_raw[pl.ds(g * 16, 16)]
            idx_p[pl.ds(g * 16, 16)] = v - lax.div(v, R) * R
            return ()

        lax.fori_loop(0, BPW // 16, to_packed, ())
        pltpu.async_copy(packed.at[idx_p], rows, sem).wait()
        pltpu.sync_copy(rows, out.at[pl.ds(base, BPW)])


_sc_gather = pl.kernel(
    _sc_gather_body,
    mesh=plsc.VectorSubcoreMesh(core_axis_name="c", subcore_axis_name="s"),
    out_type=[
        jax.ShapeDtypeStruct((B, H), jnp.int32),
        jax.ShapeDtypeStruct((B, H), jnp.int32),
    ],
    scratch_types=[
        pltpu.VMEM((BPW,), jnp.int32),
        pltpu.VMEM((BPW,), jnp.int32),
        pltpu.VMEM((BPW, H), jnp.int32),
        pltpu.SemaphoreType.DMA,
    ],
    compiler_params=pltpu.CompilerParams(use_tc_tiling_on_sc=True),
)


def _tc_towers_body(u_ref, v_ref, uid, iid, w0u, b0u, w1u, b1u,
                    w0i, b0i, w1i, b1i, o_ref):
    def decode(q_ref, ids_ref):
        q = q_ref[...]
        k = (ids_ref[...] // R)[:, None]
        w = jnp.where(k % 2 == 0, q[:, :EMB], q[:, EMB:])
        bits = jnp.where(k < 2, w & _HI, lax.shift_left(w, 16))
        return lax.bitcast_convert_type(bits, jnp.float32)

    def tower(x, W0, b0, W1, b1):
        h = lax.dot_general(x, W0[...], (((1,), (1,)), ((), ())),
                            preferred_element_type=jnp.float32)
        h = jnp.maximum(h + b0[...], 0.0)
        h = lax.dot_general(h, W1[...], (((1,), (1,)), ((), ())),
                            preferred_element_type=jnp.float32)
        return jnp.maximum(h + b1[...], 0.0)

    uo = tower(decode(u_ref, uid), w0u, b0u, w1u, b1u)
    vo = tower(decode(v_ref, iid), w0i, b0i, w1i, b1i)
    o_ref[...] = jnp.sum(uo * vo, axis=-1)


def _tc_towers(u_r, v_r, uid, iid,
               W0_u, b0_u, W1_u, b1_u, W0_i, b0_i, W1_i, b1_i):
    full = lambda shape: pl.BlockSpec(shape, lambda i: (0,) * len(shape))
    return pl.pallas_call(
        _tc_towers_body,
        grid=(B // BLK,),
        in_specs=[
            pl.BlockSpec((BLK, H), lambda i: (i, 0)),
            pl.BlockSpec((BLK, H), lambda i: (i, 0)),
            pl.BlockSpec((BLK,), lambda i: (i,)),
            pl.BlockSpec((BLK,), lambda i: (i,)),
            full((H, EMB)), full((1, H)),
            full((EMB, H)), full((1, EMB)),
            full((H, EMB)), full((1, H)),
            full((EMB, H)), full((1, EMB)),
        ],
        out_specs=pl.BlockSpec((BLK,), lambda i: (i,)),
        out_shape=jax.ShapeDtypeStruct((B,), jnp.float32),
    )(u_r, v_r, uid, iid,
      W0_u, b0_u.reshape(1, H), W1_u, b1_u.reshape(1, EMB),
      W0_i, b0_i.reshape(1, H), W1_i, b1_i.reshape(1, EMB))


def kernel(user_ids, item_ids, user_table, item_table,
           W0_u, b0_u, W1_u, b1_u, W0_i, b0_i, W1_i, b1_i):
    uid = user_ids.astype(jnp.int32)
    iid = item_ids.astype(jnp.int32)
    pu, pi = _repack(user_table.T, item_table.T)
    u_r, v_r = _sc_gather(pu, pi, uid, iid)
    return _tc_towers(u_r, v_r, uid, iid, W0_u, b0_u, W1_u, b1_u,
                      W0_i, b0_i, W1_i, b1_i)


# speedup vs baseline: 4.9572x; 1.0548x over previous
import jax
import jax.numpy as jnp
from jax import lax
from jax.experimental import pallas as pl
from jax.experimental.pallas import tpu as pltpu
from jax.experimental.pallas import tpu_sc as plsc

B = 16384
EMB = 64
H = 128
N_ROWS = 1000000
NC = 2
NS = 16
NW = NC * NS
BPW = B // NW

RBLK = 8192
NBLK = 31
R = NBLK * RBLK

BLK = 2048

_MAX_CBLK = (N_ROWS + RBLK - 1) // RBLK - 1
_HI = -65536


def _repack_body(u0, u1, u2, u3, i0, i1, i2, i3, ou_ref, oi_ref):
    def bits(tk):
        b = lax.bitcast_convert_type(tk[...], jnp.int32)
        return (b + 32768) & _HI

    for t0, t1, t2, t3, o_ref in ((u0, u1, u2, u3, ou_ref),
                                  (i0, i1, i2, i3, oi_ref)):
        c02 = bits(t0) | lax.shift_right_logical(bits(t2), 16)
        c13 = bits(t1) | lax.shift_right_logical(bits(t3), 16)
        o_ref[:, :EMB] = jnp.swapaxes(c02, 0, 1)
        o_ref[:, EMB:] = jnp.swapaxes(c13, 0, 1)


def _repack(ut, it):
    view = lambda k: pl.BlockSpec(
        (EMB, RBLK), lambda i, k=k: (0, jnp.minimum(k * NBLK + i, _MAX_CBLK)))
    out_spec = pl.BlockSpec((RBLK, H), lambda i: (i, 0))
    out_sds = jax.ShapeDtypeStruct((R, H), jnp.int32)
    return pl.pallas_call(
        _repack_body,
        grid=(NBLK,),
        in_specs=[view(0), view(1), view(2), view(3)] * 2,
        out_specs=[out_spec, out_spec],
        out_shape=[out_sds, out_sds],
    )(ut, ut, ut, ut, it, it, it, it)


def _sc_gather_body(pu, pi, uid, iid, u_out, v_out,
                    idx_raw, idx_p, rows, sem):
    wid = lax.axis_index("s") * NC + lax.axis_index("c")
    base = wid * BPW
    for ids, packed, out in ((uid, pu, u_out), (iid, pi, v_out)):
        pltpu.sync_copy(ids.at[pl.ds(base, BPW)], idx_raw)

        def to_packed(g, _):
            v = idx_raw[pl.ds(g * 16, 16)]
            idx_p[pl.ds(g * 16, 16)] = v - lax.div(v, R) * R
            return ()

        lax.fori_loop(0, BPW // 16, to_packed, ())
        pltpu.async_copy(packed.at[idx_p], rows, sem).wait()
        pltpu.sync_copy(rows, out.at[pl.ds(base, BPW)])


_sc_gather = pl.kernel(
    _sc_gather_body,
    mesh=plsc.VectorSubcoreMesh(core_axis_name="c", subcore_axis_name="s"),
    out_type=[
        jax.ShapeDtypeStruct((B, H), jnp.int32),
        jax.ShapeDtypeStruct((B, H), jnp.int32),
    ],
    scratch_types=[
        pltpu.VMEM((BPW,), jnp.int32),
        pltpu.VMEM((BPW,), jnp.int32),
        pltpu.VMEM((BPW, H), jnp.int32),
        pltpu.SemaphoreType.DMA,
    ],
    compiler_params=pltpu.CompilerParams(use_tc_tiling_on_sc=True),
)


def _tc_towers_body(u_ref, v_ref, uid, iid, w0u, b0u, w1u, b1u,
                    w0i, b0i, w1i, b1i, o_ref):
    def decode(q_ref, ids_ref):
        q = q_ref[...]
        k = (ids_ref[...] // R)[:, None]
        w = jnp.where(k % 2 == 0, q[:, :EMB], q[:, EMB:])
        bits = jnp.where(k < 2, w & _HI, lax.shift_left(w, 16))
        return lax.bitcast_convert_type(bits, jnp.float32)

    def tower(x, W0, b0, W1, b1):
        h = lax.dot_general(x, W0[...], (((1,), (1,)), ((), ())),
                            preferred_element_type=jnp.float32)
        h = jnp.maximum(h + b0[...], 0.0)
        h = lax.dot_general(h, W1[...], (((1,), (1,)), ((), ())),
                            preferred_element_type=jnp.float32)
        return jnp.maximum(h + b1[...], 0.0)

    uo = tower(decode(u_ref, uid), w0u, b0u, w1u, b1u)
    vo = tower(decode(v_ref, iid), w0i, b0i, w1i, b1i)
    o_ref[...] = jnp.sum(uo * vo, axis=-1)


def _tc_towers(u_r, v_r, uid, iid,
               W0_u, b0_u, W1_u, b1_u, W0_i, b0_i, W1_i, b1_i):
    full = lambda shape: pl.BlockSpec(shape, lambda i: (0,) * len(shape))
    return pl.pallas_call(
        _tc_towers_body,
        grid=(B // BLK,),
        in_specs=[
            pl.BlockSpec((BLK, H), lambda i: (i, 0)),
            pl.BlockSpec((BLK, H), lambda i: (i, 0)),
            pl.BlockSpec((BLK,), lambda i: (i,)),
            pl.BlockSpec((BLK,), lambda i: (i,)),
            full((H, EMB)), full((1, H)),
            full((EMB, H)), full((1, EMB)),
            full((H, EMB)), full((1, H)),
            full((EMB, H)), full((1, EMB)),
        ],
        out_specs=pl.BlockSpec((BLK,), lambda i: (i,)),
        out_shape=jax.ShapeDtypeStruct((B,), jnp.float32),
    )(u_r, v_r, uid, iid,
      W0_u, b0_u.reshape(1, H), W1_u, b1_u.reshape(1, EMB),
      W0_i, b0_i.reshape(1, H), W1_i, b1_i.reshape(1, EMB))


def kernel(user_ids, item_ids, user_table, item_table,
           W0_u, b0_u, W1_u, b1_u, W0_i, b0_i, W1_i, b1_i):
    uid = user_ids.astype(jnp.int32)
    iid = item_ids.astype(jnp.int32)
    pu, pi = _repack(user_table.T, item_table.T)
    u_r, v_r = _sc_gather(pu, pi, uid, iid)
    return _tc_towers(u_r, v_r, uid, iid, W0_u, b0_u, W1_u, b1_u,
                      W0_i, b0_i, W1_i, b1_i)
